# baseline (speedup 1.0000x reference)
import jax
import jax.numpy as jnp
from jax import lax
from jax.experimental import pallas as pl
from jax.experimental.pallas import tpu as pltpu
from jax.experimental.pallas import tpu_sc as plsc

N_NODES = 10000
N_PAD = 10240
N_EDGES = 320000
D = 128

NC = 2
NS = 16
NW = NC * NS
CHUNK = 128
E_MAIN = 10240
E_LAST = N_EDGES - E_MAIN * (NW - 1)
ROWS_MAIN = E_MAIN // CHUNK
ROWS_LAST = E_LAST // CHUNK
ROWS_PER_S = N_PAD // NS


def _sc_body(feat_hbm, src_hbm, dst_hbm, agg_out, deg_out,
             srcc0, srcc1, dstc0, dstc1, dstc2, dstc3, rows0, rows1,
             ones_v, degbuf_v,
             semg0, semg1, semis0, semis1, semid0, semid1, semid2, semid3,
             semS0, semS1, semD0, semD1, semD2, semD3, agg_sh, deg_sh):
    c = lax.axis_index("c")
    s = lax.axis_index("s")
    wid = s * NC + c
    base = wid * E_MAIN
    nrows = jnp.where(wid == NW - 1, ROWS_LAST, ROWS_MAIN)

    zeros16 = jnp.zeros((16,), jnp.float32)
    ones16 = jnp.ones((16,), jnp.float32)
    for i in range(CHUNK // 16):
        ones_v[pl.ds(i * 16, 16)] = ones16

    def zrow_body(r, carry):
        for j in range(D // 16):
            rows0[r, pl.ds(j * 16, 16)] = zeros16
        return carry

    lax.fori_loop(0, 128, zrow_body, 0)
    for i in range(ROWS_PER_S // 16):
        degbuf_v[pl.ds(i * 16, 16)] = zeros16

    for k in range(ROWS_PER_S // 128):
        pltpu.sync_copy(rows0, agg_sh.at[pl.ds(s * ROWS_PER_S + k * 128, 128)])
    pltpu.sync_copy(degbuf_v, deg_sh.at[pl.ds(s * ROWS_PER_S, ROWS_PER_S)])

    srccs = (srcc0, srcc1)
    dstcs = (dstc0, dstc1, dstc2, dstc3)
    rowss = (rows0, rows1)
    semiss = (semis0, semis1)
    semids = (semid0, semid1, semid2, semid3)
    semgs = (semg0, semg1)
    semSs = (semS0, semS1)
    semDs = (semD0, semD1, semD2, semD3)

    pltpu.sync_copy(src_hbm.at[pl.ds(base, CHUNK)], srcc0)
    pltpu.sync_copy(dst_hbm.at[pl.ds(base, CHUNK)], dstc0)
    pltpu.async_copy(src_hbm.at[pl.ds(base + CHUNK, CHUNK)], srcc1, semis1)
    pltpu.async_copy(dst_hbm.at[pl.ds(base + CHUNK, CHUNK)], dstc1, semid1)
    pltpu.async_copy(feat_hbm.at[srcc0], rows0, semg0)
    plsc.subcore_barrier()

    def chunk_body(ko, carry):
        for b in range(4):
            i = 4 * ko + b
            rb, ob = b % 2, (b + 1) % 2
            rows, semg = rowss[rb], semgs[rb]

            pltpu.make_async_copy(feat_hbm.at[srccs[rb]], rows, semg).wait()

            @pl.when(i + 1 < nrows)
            def _():
                pltpu.make_async_copy(
                    src_hbm.at[pl.ds(base + (i + 1) * CHUNK, CHUNK)],
                    srccs[ob], semiss[ob]).wait()
                pltpu.make_async_copy(
                    dst_hbm.at[pl.ds(base + (i + 1) * CHUNK, CHUNK)],
                    dstcs[(b + 1) % 4], semids[(b + 1) % 4]).wait()

                @pl.when(i >= 1)
                def _():
                    pltpu.make_async_copy(
                        rowss[ob], agg_sh.at[dstcs[(b + 3) % 4]],
                        semSs[ob]).wait()

                pltpu.async_copy(feat_hbm.at[srccs[ob]], rowss[ob], semgs[ob])

            pltpu.async_copy(rows, agg_sh.at[dstcs[b]], semSs[rb], add=True)
            pltpu.async_copy(ones_v, deg_sh.at[dstcs[b]], semDs[b], add=True)

            @pl.when(i + 2 < nrows)
            def _():
                @pl.when(i >= 2)
                def _():
                    pltpu.make_async_copy(
                        ones_v, deg_sh.at[dstcs[(b + 2) % 4]],
                        semDs[(b + 2) % 4]).wait()

                pltpu.async_copy(
                    src_hbm.at[pl.ds(base + (i + 2) * CHUNK, CHUNK)],
                    srccs[rb], semiss[rb])
                pltpu.async_copy(
                    dst_hbm.at[pl.ds(base + (i + 2) * CHUNK, CHUNK)],
                    dstcs[(b + 2) % 4], semids[(b + 2) % 4])

        return carry

    lax.fori_loop(0, nrows // 4, chunk_body, 0)

    pltpu.make_async_copy(rows0, agg_sh.at[dstc0], semS0).wait()
    pltpu.make_async_copy(rows1, agg_sh.at[dstc1], semS1).wait()
    pltpu.make_async_copy(ones_v, deg_sh.at[dstc0], semD0).wait()
    pltpu.make_async_copy(ones_v, deg_sh.at[dstc1], semD1).wait()
    pltpu.make_async_copy(ones_v, deg_sh.at[dstc2], semD2).wait()
    pltpu.make_async_copy(ones_v, deg_sh.at[dstc3], semD3).wait()
    plsc.subcore_barrier()

    for k in range(ROWS_PER_S // 128):
        r0 = s * ROWS_PER_S + k * 128
        pltpu.sync_copy(agg_sh.at[pl.ds(r0, 128)], rows0)
        pltpu.sync_copy(rows0, agg_out.at[pl.ds(c * N_PAD + r0, 128)])
    pltpu.sync_copy(deg_sh.at[pl.ds(s * ROWS_PER_S, ROWS_PER_S)], degbuf_v)
    pltpu.sync_copy(degbuf_v, deg_out.at[pl.ds(c * N_PAD + s * ROWS_PER_S, ROWS_PER_S)])


_sc_scatter = pl.kernel(
    _sc_body,
    out_type=[
        jax.ShapeDtypeStruct((NC * N_PAD, D), jnp.float32),
        jax.ShapeDtypeStruct((NC * N_PAD,), jnp.float32),
    ],
    mesh=plsc.VectorSubcoreMesh(core_axis_name="c", subcore_axis_name="s"),
    scratch_types=(
        [pltpu.VMEM((CHUNK,), jnp.int32)] * 6
        + [pltpu.VMEM((CHUNK, D), jnp.float32)] * 2
        + [pltpu.VMEM((CHUNK,), jnp.float32),
           pltpu.VMEM((ROWS_PER_S,), jnp.float32)]
        + [pltpu.SemaphoreType.DMA] * 14
        + [pltpu.VMEM_SHARED((N_PAD, D), jnp.float32),
           pltpu.VMEM_SHARED((N_PAD,), jnp.float32)]
    ),
)


def _tc_body(agg2, deg2, f, sn, w, out):
    a = agg2[...]
    d = deg2[...]
    agg = a[0] + a[1]
    deg = d[0] + d[1]
    mean = agg / jnp.maximum(deg, 1.0)
    h = jnp.where(deg > 0.0, mean, f[...])
    h = jnp.dot(h, w[...], preferred_element_type=jnp.float32)
    h = h * sn[...]
    out[...] = jnp.maximum(h, 0.0)


_BLK = 1000


def _tc_combine(agg2, deg2, feature, snorm_n, W):
    grid = (N_NODES // _BLK,)
    return pl.pallas_call(
        _tc_body,
        grid=grid,
        in_specs=[
            pl.BlockSpec((NC, _BLK, D), lambda i: (0, i, 0)),
            pl.BlockSpec((NC, _BLK, 1), lambda i: (0, i, 0)),
            pl.BlockSpec((_BLK, D), lambda i: (i, 0)),
            pl.BlockSpec((_BLK, 1), lambda i: (i, 0)),
            pl.BlockSpec((D, D), lambda i: (0, 0)),
        ],
        out_specs=pl.BlockSpec((_BLK, D), lambda i: (i, 0)),
        out_shape=jax.ShapeDtypeStruct((N_NODES, D), jnp.float32),
    )(agg2, deg2, feature, snorm_n, W)


@jax.jit
def kernel(feature, edge_index, snorm_n, W):
    src = edge_index[0]
    dst = edge_index[1]
    agg2, deg2 = _sc_scatter(feature, src, dst)
    return _tc_combine(agg2.reshape(NC, N_PAD, D), deg2.reshape(NC, N_PAD, 1),
                       feature, snorm_n, W)


# speedup vs baseline: 1.0043x; 1.0043x over previous
import jax
import jax.numpy as jnp
from jax import lax
from jax.experimental import pallas as pl
from jax.experimental.pallas import tpu as pltpu
from jax.experimental.pallas import tpu_sc as plsc

N_NODES = 10000
N_PAD = 10240
N_EDGES = 320000
D = 128

NC = 2
NS = 16
NW = NC * NS
CHUNK = 128
E_MAIN = 10240
E_LAST = N_EDGES - E_MAIN * (NW - 1)
ROWS_MAIN = E_MAIN // CHUNK
ROWS_LAST = E_LAST // CHUNK
ROWS_PER_S = N_PAD // NS


def _sc_body(feat_hbm, src_hbm, dst_hbm, agg_out, deg_out,
             srcc0, srcc1, dstc0, dstc1, dstc2, dstc3, rows0, rows1,
             ones_v, degbuf_v,
             semg0, semg1, semis0, semis1, semid0, semid1, semid2, semid3,
             semS0, semS1, semD0, semD1, semD2, semD3, agg_sh, deg_sh):
    c = lax.axis_index("c")
    s = lax.axis_index("s")
    wid = s * NC + c
    base = wid * E_MAIN
    nrows = jnp.where(wid == NW - 1, ROWS_LAST, ROWS_MAIN)

    zeros16 = jnp.zeros((16,), jnp.float32)
    ones16 = jnp.ones((16,), jnp.float32)
    for i in range(CHUNK // 16):
        ones_v[pl.ds(i * 16, 16)] = ones16

    def zrow_body(r, carry):
        for j in range(D // 16):
            rows0[r, pl.ds(j * 16, 16)] = zeros16
        return carry

    lax.fori_loop(0, 128, zrow_body, 0)
    for i in range(ROWS_PER_S // 16):
        degbuf_v[pl.ds(i * 16, 16)] = zeros16

    for k in range(ROWS_PER_S // 128):
        pltpu.sync_copy(rows0, agg_sh.at[pl.ds(s * ROWS_PER_S + k * 128, 128)])
    pltpu.sync_copy(degbuf_v, deg_sh.at[pl.ds(s * ROWS_PER_S, ROWS_PER_S)])

    srccs = (srcc0, srcc1)
    dstcs = (dstc0, dstc1, dstc2, dstc3)
    rowss = (rows0, rows1)
    semiss = (semis0, semis1)
    semids = (semid0, semid1, semid2, semid3)
    semgs = (semg0, semg1)
    semSs = (semS0, semS1)
    semDs = (semD0, semD1, semD2, semD3)

    pltpu.sync_copy(src_hbm.at[pl.ds(base, CHUNK)], srcc0)
    pltpu.sync_copy(dst_hbm.at[pl.ds(base, CHUNK)], dstc0)
    pltpu.async_copy(src_hbm.at[pl.ds(base + CHUNK, CHUNK)], srcc1, semis1)
    pltpu.async_copy(dst_hbm.at[pl.ds(base + CHUNK, CHUNK)], dstc1, semid1)
    pltpu.async_copy(feat_hbm.at[srcc0.at[pl.ds(0, CHUNK // 2)]],
                     rows0.at[pl.ds(0, CHUNK // 2)], semg0)
    pltpu.async_copy(feat_hbm.at[srcc0.at[pl.ds(CHUNK // 2, CHUNK // 2)]],
                     rows0.at[pl.ds(CHUNK // 2, CHUNK // 2)], semg0)
    plsc.subcore_barrier()

    def chunk_body(ko, carry):
        for b in range(4):
            i = 4 * ko + b
            rb, ob = b % 2, (b + 1) % 2
            rows, semg = rowss[rb], semgs[rb]

            pltpu.make_async_copy(
                feat_hbm.at[srccs[rb].at[pl.ds(0, CHUNK // 2)]],
                rows.at[pl.ds(0, CHUNK // 2)], semg).wait()
            pltpu.make_async_copy(
                feat_hbm.at[srccs[rb].at[pl.ds(CHUNK // 2, CHUNK // 2)]],
                rows.at[pl.ds(CHUNK // 2, CHUNK // 2)], semg).wait()

            @pl.when(i + 1 < nrows)
            def _():
                pltpu.make_async_copy(
                    src_hbm.at[pl.ds(base + (i + 1) * CHUNK, CHUNK)],
                    srccs[ob], semiss[ob]).wait()
                pltpu.make_async_copy(
                    dst_hbm.at[pl.ds(base + (i + 1) * CHUNK, CHUNK)],
                    dstcs[(b + 1) % 4], semids[(b + 1) % 4]).wait()

                @pl.when(i >= 1)
                def _():
                    pltpu.make_async_copy(
                        rowss[ob], agg_sh.at[dstcs[(b + 3) % 4]],
                        semSs[ob]).wait()

                pltpu.async_copy(
                    feat_hbm.at[srccs[ob].at[pl.ds(0, CHUNK // 2)]],
                    rowss[ob].at[pl.ds(0, CHUNK // 2)], semgs[ob])
                pltpu.async_copy(
                    feat_hbm.at[srccs[ob].at[pl.ds(CHUNK // 2, CHUNK // 2)]],
                    rowss[ob].at[pl.ds(CHUNK // 2, CHUNK // 2)], semgs[ob])

            pltpu.async_copy(rows, agg_sh.at[dstcs[b]], semSs[rb], add=True)
            pltpu.async_copy(ones_v, deg_sh.at[dstcs[b]], semDs[b], add=True)

            @pl.when(i + 2 < nrows)
            def _():
                @pl.when(i >= 2)
                def _():
                    pltpu.make_async_copy(
                        ones_v, deg_sh.at[dstcs[(b + 2) % 4]],
                        semDs[(b + 2) % 4]).wait()

                pltpu.async_copy(
                    src_hbm.at[pl.ds(base + (i + 2) * CHUNK, CHUNK)],
                    srccs[rb], semiss[rb])
                pltpu.async_copy(
                    dst_hbm.at[pl.ds(base + (i + 2) * CHUNK, CHUNK)],
                    dstcs[(b + 2) % 4], semids[(b + 2) % 4])

        return carry

    lax.fori_loop(0, nrows // 4, chunk_body, 0)

    pltpu.make_async_copy(rows0, agg_sh.at[dstc0], semS0).wait()
    pltpu.make_async_copy(rows1, agg_sh.at[dstc1], semS1).wait()
    pltpu.make_async_copy(ones_v, deg_sh.at[dstc0], semD0).wait()
    pltpu.make_async_copy(ones_v, deg_sh.at[dstc1], semD1).wait()
    pltpu.make_async_copy(ones_v, deg_sh.at[dstc2], semD2).wait()
    pltpu.make_async_copy(ones_v, deg_sh.at[dstc3], semD3).wait()
    plsc.subcore_barrier()

    for k in range(ROWS_PER_S // 128):
        r0 = s * ROWS_PER_S + k * 128
        pltpu.sync_copy(agg_sh.at[pl.ds(r0, 128)], rows0)
        pltpu.sync_copy(rows0, agg_out.at[pl.ds(c * N_PAD + r0, 128)])
    pltpu.sync_copy(deg_sh.at[pl.ds(s * ROWS_PER_S, ROWS_PER_S)], degbuf_v)
    pltpu.sync_copy(degbuf_v, deg_out.at[pl.ds(c * N_PAD + s * ROWS_PER_S, ROWS_PER_S)])


_sc_scatter = pl.kernel(
    _sc_body,
    out_type=[
        jax.ShapeDtypeStruct((NC * N_PAD, D), jnp.float32),
        jax.ShapeDtypeStruct((NC * N_PAD,), jnp.float32),
    ],
    mesh=plsc.VectorSubcoreMesh(core_axis_name="c", subcore_axis_name="s"),
    scratch_types=(
        [pltpu.VMEM((CHUNK,), jnp.int32)] * 6
        + [pltpu.VMEM((CHUNK, D), jnp.float32)] * 2
        + [pltpu.VMEM((CHUNK,), jnp.float32),
           pltpu.VMEM((ROWS_PER_S,), jnp.float32)]
        + [pltpu.SemaphoreType.DMA] * 14
        + [pltpu.VMEM_SHARED((N_PAD, D), jnp.float32),
           pltpu.VMEM_SHARED((N_PAD,), jnp.float32)]
    ),
)


def _tc_body(agg2, deg2, f, sn, w, out):
    a = agg2[...]
    d = deg2[...]
    agg = a[0] + a[1]
    deg = d[0] + d[1]
    mean = agg / jnp.maximum(deg, 1.0)
    h = jnp.where(deg > 0.0, mean, f[...])
    h = jnp.dot(h, w[...], preferred_element_type=jnp.float32)
    h = h * sn[...]
    out[...] = jnp.maximum(h, 0.0)


_BLK = 1000


def _tc_combine(agg2, deg2, feature, snorm_n, W):
    grid = (N_NODES // _BLK,)
    return pl.pallas_call(
        _tc_body,
        grid=grid,
        in_specs=[
            pl.BlockSpec((NC, _BLK, D), lambda i: (0, i, 0)),
            pl.BlockSpec((NC, _BLK, 1), lambda i: (0, i, 0)),
            pl.BlockSpec((_BLK, D), lambda i: (i, 0)),
            pl.BlockSpec((_BLK, 1), lambda i: (i, 0)),
            pl.BlockSpec((D, D), lambda i: (0, 0)),
        ],
        out_specs=pl.BlockSpec((_BLK, D), lambda i: (i, 0)),
        out_shape=jax.ShapeDtypeStruct((N_NODES, D), jnp.float32),
    )(agg2, deg2, feature, snorm_n, W)


@jax.jit
def kernel(feature, edge_index, snorm_n, W):
    src = edge_index[0]
    dst = edge_index[1]
    agg2, deg2 = _sc_scatter(feature, src, dst)
    return _tc_combine(agg2.reshape(NC, N_PAD, D), deg2.reshape(NC, N_PAD, 1),
                       feature, snorm_n, W)
